# baseline (device time: 66232 ns/iter reference)
import jax
import jax.numpy as jnp
from jax import lax
from jax.experimental import pallas as pl
from jax.experimental.pallas import tpu as pltpu

N_DEV = 4
FP8 = jnp.float8_e5m2


def _quant_fp8(a, blk_rows=512):
    m, n = a.shape

    def body(a_ref, o_ref):
        o_ref[...] = a_ref[...].astype(FP8)

    return pl.pallas_call(
        body,
        grid=(m // blk_rows,),
        in_specs=[pl.BlockSpec((blk_rows, n), lambda i: (i, 0))],
        out_specs=pl.BlockSpec((blk_rows, n), lambda i: (i, 0)),
        out_shape=jax.ShapeDtypeStruct((m, n), FP8),
    )(a)


def kernel(x, w_mat, scale_x, scale_w):
    m_glob, k_shard = x.shape
    k_glob, n = w_mat.shape
    m_per = m_glob // N_DEV

    wq = _quant_fp8(w_mat)

    def body(x_ref, w_ref, sx_ref, sw_ref, out_ref,
             xq_ref, recv_ref, send_sems, recv_sems):
        my = lax.axis_index("i")

        xq_ref[...] = x_ref[...].astype(FP8)

        bsem = pltpu.get_barrier_semaphore()
        for k in range(1, N_DEV):
            peer = lax.rem(my + k, N_DEV)
            pl.semaphore_signal(
                bsem, inc=1,
                device_id=(peer,), device_id_type=pl.DeviceIdType.MESH,
            )
        pl.semaphore_wait(bsem, N_DEV - 1)

        sends = []
        for k in range(1, N_DEV):
            dst = lax.rem(my + k, N_DEV)
            rdma = pltpu.make_async_remote_copy(
                src_ref=xq_ref.at[pl.ds(dst * m_per, m_per), :],
                dst_ref=recv_ref.at[k - 1],
                send_sem=send_sems.at[k - 1],
                recv_sem=recv_sems.at[k - 1],
                device_id=(dst,),
                device_id_type=pl.DeviceIdType.MESH,
            )
            rdma.start()
            sends.append(rdma)

        out_ref[...] = jnp.dot(
            xq_ref[pl.ds(my * m_per, m_per), :],
            w_ref[pl.ds(my * k_shard, k_shard), :],
            preferred_element_type=jnp.float32,
        )

        for k in range(1, N_DEV):
            src = lax.rem(my - k + N_DEV, N_DEV)
            sends[k - 1].wait_recv()
            out_ref[...] += jnp.dot(
                recv_ref[k - 1],
                w_ref[pl.ds(src * k_shard, k_shard), :],
                preferred_element_type=jnp.float32,
            )

        s = sx_ref[0] * sw_ref[0]
        y = out_ref[...] * s
        z = jnp.clip(y, -60.0, 60.0)
        out_ref[...] = y / (1.0 + jnp.exp(-z))

        for rdma in sends:
            rdma.wait_send()

    return pl.pallas_call(
        body,
        out_shape=jax.ShapeDtypeStruct((m_per, n), jnp.float32),
        in_specs=[
            pl.BlockSpec(memory_space=pltpu.VMEM),
            pl.BlockSpec(memory_space=pltpu.VMEM),
            pl.BlockSpec(memory_space=pltpu.SMEM),
            pl.BlockSpec(memory_space=pltpu.SMEM),
        ],
        out_specs=pl.BlockSpec(memory_space=pltpu.VMEM),
        scratch_shapes=[
            pltpu.VMEM((m_glob, k_shard), FP8),
            pltpu.VMEM((N_DEV - 1, m_per, k_shard), FP8),
            pltpu.SemaphoreType.DMA((N_DEV - 1,)),
            pltpu.SemaphoreType.DMA((N_DEV - 1,)),
        ],
        compiler_params=pltpu.CompilerParams(
            collective_id=0,
            vmem_limit_bytes=100 * 1024 * 1024,
        ),
    )(x, wq, scale_x, scale_w)


# device time: 50059 ns/iter; 1.3231x vs baseline; 1.3231x over previous
import jax
import jax.numpy as jnp
from jax import lax
from jax.experimental import pallas as pl
from jax.experimental.pallas import tpu as pltpu

N_DEV = 4
FP8 = jnp.float8_e5m2


def kernel(x, w_mat, scale_x, scale_w):
    m_glob, k_shard = x.shape
    k_glob, n = w_mat.shape
    m_per = m_glob // N_DEV

    K_ORDER = (1, 3, 2)

    def body(x_hbm, w_hbm, sx_ref, sw_ref, out_ref,
             xstage, xqs, wstage, wq, recv_ref,
             xload_sems, wload_sems, send_sems, recv_sems):
        my = lax.axis_index("i")

        def xload(idx, slot):
            b = lax.rem(my + 1 + idx, N_DEV)
            return pltpu.make_async_copy(
                x_hbm.at[pl.ds(b * m_per, m_per), :],
                xstage.at[slot], xload_sems.at[slot])

        def wload(j, slot):
            src = lax.rem(my if j == 0 else my - K_ORDER[j - 1] + N_DEV,
                          N_DEV)
            return pltpu.make_async_copy(
                w_hbm.at[pl.ds(src * k_shard, k_shard), :],
                wstage.at[slot], wload_sems.at[slot])

        xload(0, 0).start()
        wload(0, 0).start()

        bsem = pltpu.get_barrier_semaphore()
        for k in range(1, N_DEV):
            peer = lax.rem(my + k, N_DEV)
            pl.semaphore_signal(
                bsem, inc=1,
                device_id=(peer,), device_id_type=pl.DeviceIdType.MESH,
            )
        pl.semaphore_wait(bsem, N_DEV - 1)

        sends = []
        for idx in range(N_DEV):
            if idx + 1 < N_DEV:
                xload(idx + 1, (idx + 1) % 2).start()
            xload(idx, idx % 2).wait()
            xqs[idx] = xstage[idx % 2].astype(FP8)
            if idx < 3:
                k = idx + 1
                dst = lax.rem(my + k, N_DEV)
                rdma = pltpu.make_async_remote_copy(
                    src_ref=xqs.at[idx],
                    dst_ref=recv_ref.at[k - 1],
                    send_sem=send_sems.at[k - 1],
                    recv_sem=recv_sems.at[k - 1],
                    device_id=(dst,),
                    device_id_type=pl.DeviceIdType.MESH,
                )
                rdma.start()
                sends.append(rdma)

        for j in range(N_DEV):
            if j + 1 < N_DEV:
                wload(j + 1, (j + 1) % 2).start()
            wload(j, j % 2).wait()
            wq[j % 2] = wstage[j % 2].astype(FP8)
            if j == 0:
                out_ref[...] = jnp.dot(
                    xqs[3], wq[0], preferred_element_type=jnp.float32)
            else:
                k = K_ORDER[j - 1]
                sends[k - 1].wait_recv()
                out_ref[...] += jnp.dot(
                    recv_ref[k - 1], wq[j % 2],
                    preferred_element_type=jnp.float32)

        s = sx_ref[0] * sw_ref[0]
        y = out_ref[...] * s
        z = jnp.clip(y, -60.0, 60.0)
        out_ref[...] = y / (1.0 + jnp.exp(-z))

        for rdma in sends:
            rdma.wait_send()

    return pl.pallas_call(
        body,
        out_shape=jax.ShapeDtypeStruct((m_per, n), jnp.float32),
        in_specs=[
            pl.BlockSpec(memory_space=pl.ANY),
            pl.BlockSpec(memory_space=pl.ANY),
            pl.BlockSpec(memory_space=pltpu.SMEM),
            pl.BlockSpec(memory_space=pltpu.SMEM),
        ],
        out_specs=pl.BlockSpec(memory_space=pltpu.VMEM),
        scratch_shapes=[
            pltpu.VMEM((2, m_per, k_shard), jnp.float32),
            pltpu.VMEM((N_DEV, m_per, k_shard), FP8),
            pltpu.VMEM((2, k_shard, n), jnp.float32),
            pltpu.VMEM((2, k_shard, n), FP8),
            pltpu.VMEM((N_DEV - 1, m_per, k_shard), FP8),
            pltpu.SemaphoreType.DMA((2,)),
            pltpu.SemaphoreType.DMA((2,)),
            pltpu.SemaphoreType.DMA((N_DEV - 1,)),
            pltpu.SemaphoreType.DMA((N_DEV - 1,)),
        ],
        compiler_params=pltpu.CompilerParams(
            collective_id=0,
            vmem_limit_bytes=100 * 1024 * 1024,
        ),
    )(x, w_mat, scale_x, scale_w)


# device time: 47212 ns/iter; 1.4029x vs baseline; 1.0603x over previous
import jax
import jax.numpy as jnp
from jax import lax
from jax.experimental import pallas as pl
from jax.experimental.pallas import tpu as pltpu

N_DEV = 4
FP8 = jnp.float8_e5m2

XK = (2, 1, 3, 0)
K_ORDER = (1, 3, 2)


def kernel(x, w_mat, scale_x, scale_w):
    m_glob, k_shard = x.shape
    k_glob, n = w_mat.shape
    m_per = m_glob // N_DEV

    def body(x_hbm, w_hbm, sx_ref, sw_ref, out_hbm,
             xstage, xqs, wstage, wq, recv_ref, acc,
             xload_sems, wload_sems, send_sems, recv_sems, out_sems):
        my = lax.axis_index("i")

        def xload(idx, slot):
            b = lax.rem(my + XK[idx], N_DEV)
            return pltpu.make_async_copy(
                x_hbm.at[pl.ds(b * m_per, m_per), :],
                xstage.at[slot], xload_sems.at[slot])

        def wload(j, slot):
            src = lax.rem(my if j == 0 else my - K_ORDER[j - 1] + N_DEV,
                          N_DEV)
            return pltpu.make_async_copy(
                w_hbm.at[pl.ds(src * k_shard, k_shard), :],
                wstage.at[slot], wload_sems.at[slot])

        xload(0, 0).start()

        bsem = pltpu.get_barrier_semaphore()
        for k in range(1, N_DEV):
            peer = lax.rem(my + k, N_DEV)
            pl.semaphore_signal(
                bsem, inc=1,
                device_id=(peer,), device_id_type=pl.DeviceIdType.MESH,
            )
        pl.semaphore_wait(bsem, N_DEV - 1)

        sends = []
        for idx in range(N_DEV):
            if idx + 1 < N_DEV:
                xload(idx + 1, (idx + 1) % 2).start()
            xload(idx, idx % 2).wait()
            xqs[idx] = xstage[idx % 2].astype(FP8)
            if idx < 3:
                k = XK[idx]
                dst = lax.rem(my + k, N_DEV)
                rdma = pltpu.make_async_remote_copy(
                    src_ref=xqs.at[idx],
                    dst_ref=recv_ref.at[k - 1],
                    send_sem=send_sems.at[k - 1],
                    recv_sem=recv_sems.at[k - 1],
                    device_id=(dst,),
                    device_id_type=pl.DeviceIdType.MESH,
                )
                rdma.start()
                sends.append((k, rdma))

        wload(0, 0).start()
        wload(1, 1).start()
        rdma_by_k = dict(sends)
        for j in range(N_DEV - 1):
            wload(j, j % 2).wait()
            wq[j % 2] = wstage[j % 2].astype(FP8)
            if j + 2 < N_DEV:
                wload(j + 2, j % 2).start()
            if j == 0:
                acc[...] = jnp.dot(
                    xqs[3], wq[0], preferred_element_type=jnp.float32)
            else:
                k = K_ORDER[j - 1]
                rdma_by_k[k].wait_recv()
                acc[...] += jnp.dot(
                    recv_ref[k - 1], wq[j % 2],
                    preferred_element_type=jnp.float32)

        wload(3, 1).wait()
        wq[1] = wstage[1].astype(FP8)
        k = K_ORDER[2]
        rdma_by_k[k].wait_recv()
        half = n // 2
        s = sx_ref[0] * sw_ref[0]
        odmas = []
        for h in range(2):
            lo, hi = h * half, (h + 1) * half
            y = (acc[:, lo:hi] + jnp.dot(
                recv_ref[k - 1], wq[1, :, lo:hi],
                preferred_element_type=jnp.float32)) * s
            z = jnp.clip(y, -60.0, 60.0)
            acc[:, lo:hi] = y / (1.0 + jnp.exp(-z))
            odma = pltpu.make_async_copy(
                acc.at[:, lo:hi], out_hbm.at[:, lo:hi], out_sems.at[h])
            odma.start()
            odmas.append(odma)

        for odma in odmas:
            odma.wait()
        for _, rdma in sends:
            rdma.wait_send()

    return pl.pallas_call(
        body,
        out_shape=jax.ShapeDtypeStruct((m_per, n), jnp.float32),
        in_specs=[
            pl.BlockSpec(memory_space=pl.ANY),
            pl.BlockSpec(memory_space=pl.ANY),
            pl.BlockSpec(memory_space=pltpu.SMEM),
            pl.BlockSpec(memory_space=pltpu.SMEM),
        ],
        out_specs=pl.BlockSpec(memory_space=pl.ANY),
        scratch_shapes=[
            pltpu.VMEM((2, m_per, k_shard), jnp.float32),
            pltpu.VMEM((N_DEV, m_per, k_shard), FP8),
            pltpu.VMEM((2, k_shard, n), jnp.float32),
            pltpu.VMEM((2, k_shard, n), FP8),
            pltpu.VMEM((N_DEV - 1, m_per, k_shard), FP8),
            pltpu.VMEM((m_per, n), jnp.float32),
            pltpu.SemaphoreType.DMA((2,)),
            pltpu.SemaphoreType.DMA((2,)),
            pltpu.SemaphoreType.DMA((N_DEV - 1,)),
            pltpu.SemaphoreType.DMA((N_DEV - 1,)),
            pltpu.SemaphoreType.DMA((2,)),
        ],
        compiler_params=pltpu.CompilerParams(
            collective_id=0,
            vmem_limit_bytes=100 * 1024 * 1024,
        ),
    )(x, w_mat, scale_x, scale_w)


# device time: 47078 ns/iter; 1.4069x vs baseline; 1.0028x over previous
import jax
import jax.numpy as jnp
from jax import lax
from jax.experimental import pallas as pl
from jax.experimental.pallas import tpu as pltpu

N_DEV = 4
FP8 = jnp.float8_e5m2

XK = (2, 1, 3, 0)
K_ORDER = (1, 3, 2)


def kernel(x, w_mat, scale_x, scale_w):
    m_glob, k_shard = x.shape
    k_glob, n = w_mat.shape
    m_per = m_glob // N_DEV

    def body(x_hbm, w_hbm, sx_ref, sw_ref, out_hbm,
             xstage, xqs, wstage, wq, recv_ref, acc,
             xload_sems, wload_sems, send_sems, recv_sems, out_sems):
        my = lax.axis_index("i")

        def xload(idx, slot):
            b = lax.rem(my + XK[idx], N_DEV)
            return pltpu.make_async_copy(
                x_hbm.at[pl.ds(b * m_per, m_per), :],
                xstage.at[slot], xload_sems.at[slot])

        def wload(j, slot):
            src = lax.rem(my if j == 0 else my - K_ORDER[j - 1] + N_DEV,
                          N_DEV)
            return pltpu.make_async_copy(
                w_hbm.at[pl.ds(src * k_shard, k_shard), :],
                wstage.at[slot], wload_sems.at[slot])

        xload(0, 0).start()

        bsem = pltpu.get_barrier_semaphore()
        for k in range(1, N_DEV):
            peer = lax.rem(my + k, N_DEV)
            pl.semaphore_signal(
                bsem, inc=1,
                device_id=(peer,), device_id_type=pl.DeviceIdType.MESH,
            )
        pl.semaphore_wait(bsem, N_DEV - 1)

        sends = []
        for idx in range(N_DEV):
            xload(idx, idx % 2).wait()
            if idx + 1 < N_DEV:
                xload(idx + 1, (idx + 1) % 2).start()
            xqs[idx] = xstage[idx % 2].astype(FP8)
            if idx < 3:
                k = XK[idx]
                dst = lax.rem(my + k, N_DEV)
                rdma = pltpu.make_async_remote_copy(
                    src_ref=xqs.at[idx],
                    dst_ref=recv_ref.at[k - 1],
                    send_sem=send_sems.at[k - 1],
                    recv_sem=recv_sems.at[k - 1],
                    device_id=(dst,),
                    device_id_type=pl.DeviceIdType.MESH,
                )
                rdma.start()
                sends.append((k, rdma))

        wload(0, 0).start()
        wload(1, 1).start()
        rdma_by_k = dict(sends)
        for j in range(N_DEV - 1):
            wload(j, j % 2).wait()
            wq[j % 2] = wstage[j % 2].astype(FP8)
            if j + 2 < N_DEV:
                wload(j + 2, j % 2).start()
            if j == 0:
                acc[...] = jnp.dot(
                    xqs[3], wq[0], preferred_element_type=jnp.float32)
            else:
                k = K_ORDER[j - 1]
                rdma_by_k[k].wait_recv()
                acc[...] += jnp.dot(
                    recv_ref[k - 1], wq[j % 2],
                    preferred_element_type=jnp.float32)

        wload(3, 1).wait()
        wq[1] = wstage[1].astype(FP8)
        k = K_ORDER[2]
        rdma_by_k[k].wait_recv()
        half = n // 2
        s = sx_ref[0] * sw_ref[0]
        odmas = []
        for h in range(2):
            lo, hi = h * half, (h + 1) * half
            y = (acc[:, lo:hi] + jnp.dot(
                recv_ref[k - 1], wq[1, :, lo:hi],
                preferred_element_type=jnp.float32)) * s
            z = jnp.clip(y, -60.0, 60.0)
            acc[:, lo:hi] = y / (1.0 + jnp.exp(-z))
            odma = pltpu.make_async_copy(
                acc.at[:, lo:hi], out_hbm.at[:, lo:hi], out_sems.at[h])
            odma.start()
            odmas.append(odma)

        for odma in odmas:
            odma.wait()
        for _, rdma in sends:
            rdma.wait_send()

    return pl.pallas_call(
        body,
        out_shape=jax.ShapeDtypeStruct((m_per, n), jnp.float32),
        in_specs=[
            pl.BlockSpec(memory_space=pl.ANY),
            pl.BlockSpec(memory_space=pl.ANY),
            pl.BlockSpec(memory_space=pltpu.SMEM),
            pl.BlockSpec(memory_space=pltpu.SMEM),
        ],
        out_specs=pl.BlockSpec(memory_space=pl.ANY),
        scratch_shapes=[
            pltpu.VMEM((2, m_per, k_shard), jnp.float32),
            pltpu.VMEM((N_DEV, m_per, k_shard), FP8),
            pltpu.VMEM((2, k_shard, n), jnp.float32),
            pltpu.VMEM((2, k_shard, n), FP8),
            pltpu.VMEM((N_DEV - 1, m_per, k_shard), FP8),
            pltpu.VMEM((m_per, n), jnp.float32),
            pltpu.SemaphoreType.DMA((2,)),
            pltpu.SemaphoreType.DMA((2,)),
            pltpu.SemaphoreType.DMA((N_DEV - 1,)),
            pltpu.SemaphoreType.DMA((N_DEV - 1,)),
            pltpu.SemaphoreType.DMA((2,)),
        ],
        compiler_params=pltpu.CompilerParams(
            collective_id=0,
            vmem_limit_bytes=100 * 1024 * 1024,
        ),
    )(x, w_mat, scale_x, scale_w)
